# baseline (speedup 1.0000x reference)
import functools

import jax
import jax.numpy as jnp
from jax.experimental import pallas as pl

B = 8192
D = 2048
K = 8
POOL = 64
BLK = 512


def _fused_kernel(x_ref, idx_ref, w_ref, pool_ref, out_ref):
    x = x_ref[...]
    pool = pool_ref[...]
    idx = idx_ref[...]
    w = w_ref[...]

    p = jax.lax.dot_general(
        x, pool, (((1,), (1,)), ((), ())), preferred_element_type=jnp.float32
    )
    a = 0.5 * p * (1.0 + jax.lax.erf(p * 0.7071067811865476))

    col = jax.lax.broadcasted_iota(jnp.int32, (BLK, POOL), 1)
    t = jnp.zeros((BLK, POOL), dtype=jnp.float32)
    for k in range(K):
        t = t + jnp.where(col == idx[:, k][:, None], w[:, k][:, None], 0.0)

    c = t * a
    out = jax.lax.dot_general(
        c, pool, (((1,), (0,)), ((), ())), preferred_element_type=jnp.float32
    )
    out_ref[...] = x + out


@jax.jit
def kernel(x, indices, weights, pool):
    indices = indices.astype(jnp.int32)
    grid = (B // BLK,)
    return pl.pallas_call(
        _fused_kernel,
        grid=grid,
        in_specs=[
            pl.BlockSpec((BLK, D), lambda i: (i, 0)),
            pl.BlockSpec((BLK, K), lambda i: (i, 0)),
            pl.BlockSpec((BLK, K), lambda i: (i, 0)),
            pl.BlockSpec((POOL, D), lambda i: (0, 0)),
        ],
        out_specs=pl.BlockSpec((BLK, D), lambda i: (i, 0)),
        out_shape=jax.ShapeDtypeStruct((B, D), jnp.float32),
    )(x, indices, weights, pool)


# speedup vs baseline: 1.0418x; 1.0418x over previous
import functools

import jax
import jax.numpy as jnp
from jax.experimental import pallas as pl

B = 8192
D = 2048
K = 8
POOL = 64
BLK = 1024


def _fused_kernel(x_ref, idx_ref, w_ref, pool_ref, out_ref):
    x = x_ref[...]
    pool = pool_ref[...]
    idx = idx_ref[...]
    w = w_ref[...]

    xb = x.astype(jnp.bfloat16)
    poolb = pool.astype(jnp.bfloat16)
    p = jax.lax.dot_general(
        xb, poolb, (((1,), (1,)), ((), ())), preferred_element_type=jnp.float32
    )
    a = 0.5 * p * (1.0 + jax.lax.erf(p * 0.7071067811865476))

    col = jax.lax.broadcasted_iota(jnp.int32, (BLK, POOL), 1)
    t = jnp.zeros((BLK, POOL), dtype=jnp.float32)
    for k in range(K):
        t = t + jnp.where(col == idx[:, k][:, None], w[:, k][:, None], 0.0)

    c = (t * a).astype(jnp.bfloat16)
    out = jax.lax.dot_general(
        c, poolb, (((1,), (0,)), ((), ())), preferred_element_type=jnp.float32
    )
    out_ref[...] = x + out


@jax.jit
def kernel(x, indices, weights, pool):
    indices = indices.astype(jnp.int32)
    grid = (B // BLK,)
    return pl.pallas_call(
        _fused_kernel,
        grid=grid,
        in_specs=[
            pl.BlockSpec((BLK, D), lambda i: (i, 0)),
            pl.BlockSpec((BLK, K), lambda i: (i, 0)),
            pl.BlockSpec((BLK, K), lambda i: (i, 0)),
            pl.BlockSpec((POOL, D), lambda i: (0, 0)),
        ],
        out_specs=pl.BlockSpec((BLK, D), lambda i: (i, 0)),
        out_shape=jax.ShapeDtypeStruct((B, D), jnp.float32),
    )(x, indices, weights, pool)


# speedup vs baseline: 1.1365x; 1.0909x over previous
import jax
import jax.numpy as jnp
from jax.experimental import pallas as pl

B = 8192
D = 2048
K = 8
POOL = 64
BLK = 1024


def _fused_kernel(x_ref, iw_ref, pool_ref, out_ref):
    x = x_ref[...]
    pool = pool_ref[...]
    w = iw_ref[:, :K]
    idxf = iw_ref[:, K:]

    xb = x.astype(jnp.bfloat16)
    poolb = pool.astype(jnp.bfloat16)
    p = jax.lax.dot_general(
        xb, poolb, (((1,), (1,)), ((), ())), preferred_element_type=jnp.float32
    )
    a = 0.5 * p * (1.0 + jax.lax.erf(p * 0.7071067811865476))

    idx = idxf.astype(jnp.int32)
    col = jax.lax.broadcasted_iota(jnp.int32, (BLK, POOL), 1)
    t = jnp.zeros((BLK, POOL), dtype=jnp.float32)
    for k in range(K):
        t = t + jnp.where(col == idx[:, k][:, None], w[:, k][:, None], 0.0)

    c = (t * a).astype(jnp.bfloat16)
    out = jax.lax.dot_general(
        c, poolb, (((1,), (0,)), ((), ())), preferred_element_type=jnp.float32
    )
    out_ref[...] = x + out


@jax.jit
def kernel(x, indices, weights, pool):
    iw = jnp.concatenate([weights, indices.astype(jnp.float32)], axis=1)
    grid = (B // BLK,)
    return pl.pallas_call(
        _fused_kernel,
        grid=grid,
        in_specs=[
            pl.BlockSpec((BLK, D), lambda i: (i, 0)),
            pl.BlockSpec((BLK, 2 * K), lambda i: (i, 0)),
            pl.BlockSpec((POOL, D), lambda i: (0, 0)),
        ],
        out_specs=pl.BlockSpec((BLK, D), lambda i: (i, 0)),
        out_shape=jax.ShapeDtypeStruct((B, D), jnp.float32),
    )(x, iw, pool)
